# baseline (speedup 1.0000x reference)
import functools

import jax
import jax.numpy as jnp
from jax import lax
from jax.experimental import pallas as pl
from jax.experimental.pallas import tpu as pltpu
from jax.experimental.pallas import tpu_sc as plsc

N_USERS = 25000
N_ITEMS = 25000
N_NODES = N_USERS + N_ITEMS
N_LAYERS = 3
D = 64

NC = 2
NS = 16
HALF = N_NODES // NC
ROWS_PER_TILE = 1568
ACC_ROWS = ROWS_PER_TILE * NS
DUMMY = HALF
LAST_ROWS = HALF - (NS - 1) * ROWS_PER_TILE

SUB = 128
NBUF = 6
NROW = 3

B_LOAD = 1792
FLUSH = 1024
STG = FLUSH + 32
D_APP = 896
CAP = 52224


def _part_body(col1, row1, wt1, col_c, row_c, wt_c, counts,
               ldc, ldr, ldw, stc, st_r, stw, cntv):
    c = lax.axis_index("c")
    s = lax.axis_index("s")
    node_base = c * HALF
    span = col1.shape[0] // NS
    tbase = s * span
    wid = c * NS + s

    def flush_maybe(pos, off):
        def do_flush(args):
            pos, off = args
            pltpu.sync_copy(stc.at[pl.ds(0, FLUSH)],
                            col_c.at[wid, pl.ds(pl.multiple_of(off, FLUSH), FLUSH)])
            pltpu.sync_copy(st_r.at[pl.ds(0, FLUSH)],
                            row_c.at[wid, pl.ds(pl.multiple_of(off, FLUSH), FLUSH)])
            pltpu.sync_copy(stw.at[pl.ds(0, FLUSH)],
                            wt_c.at[wid, pl.ds(pl.multiple_of(off, FLUSH), FLUSH)])
            stc[pl.ds(0, 16)] = stc[pl.ds(FLUSH, 16)]
            st_r[pl.ds(0, 16)] = st_r[pl.ds(FLUSH, 16)]
            stw[pl.ds(0, 16)] = stw[pl.ds(FLUSH, 16)]
            return (pos - FLUSH, off + FLUSH)

        return lax.cond(pos >= FLUSH, do_flush, lambda a: a, (pos, off))

    def blk(ib, carry):
        pltpu.sync_copy(col1.at[pl.ds(tbase + ib * B_LOAD, B_LOAD)], ldc)
        pltpu.sync_copy(row1.at[pl.ds(tbase + ib * B_LOAD, B_LOAD)], ldr)
        pltpu.sync_copy(wt1.at[pl.ds(tbase + ib * B_LOAD, B_LOAD)], ldw)

        def grp(g, pc):
            pos, off = pc
            sl = pl.ds(g * 16, 16)
            loc = ldr[sl] - node_base
            m = (loc >= 0) & (loc < HALF)
            plsc.store_compressed(stc.at[pl.ds(pos, 16)], ldc[sl], mask=m)
            plsc.store_compressed(st_r.at[pl.ds(pos, 16)], loc, mask=m)
            plsc.store_compressed(stw.at[pl.ds(pos, 16)], ldw[sl], mask=m)
            pos = pos + jnp.sum(m.astype(jnp.int32))
            return flush_maybe(pos, off)

        return lax.fori_loop(0, B_LOAD // 16, grp, carry)

    pos, off = lax.fori_loop(0, span // B_LOAD, blk,
                             (jnp.int32(0), jnp.int32(0)))

    cnt = off + pos
    cntv[pl.ds(0, 16)] = jnp.full((16,), cnt, jnp.int32)
    pltpu.sync_copy(cntv, counts.at[wid])

    dumc = jnp.zeros((16,), jnp.int32)
    dumr = jnp.full((16,), DUMMY, jnp.int32)
    dumw = jnp.zeros((16,), jnp.float32)

    def app(g, pc):
        pos, off = pc
        stc[pl.ds(pos, 16)] = dumc
        st_r[pl.ds(pos, 16)] = dumr
        stw[pl.ds(pos, 16)] = dumw
        return flush_maybe(pos + 16, off)

    pos, off = lax.fori_loop(0, D_APP // 16, app, (pos, off))
    pltpu.sync_copy(stc.at[pl.ds(0, FLUSH)], col_c.at[wid, pl.ds(pl.multiple_of(off, FLUSH), FLUSH)])
    pltpu.sync_copy(st_r.at[pl.ds(0, FLUSH)], row_c.at[wid, pl.ds(pl.multiple_of(off, FLUSH), FLUSH)])
    pltpu.sync_copy(stw.at[pl.ds(0, FLUSH)], wt_c.at[wid, pl.ds(pl.multiple_of(off, FLUSH), FLUSH)])


def _layer_body(table, col_c, row_c, wt_c, counts, out,
                acc, colv, rowv, wtv, rows_in, cntv,
                sem_i, sem_g, sem_s):
    c = lax.axis_index("c")
    s = lax.axis_index("s")
    node_base = c * HALF
    wid = c * NS + s
    zero16 = jnp.zeros((16,), jnp.float32)

    pltpu.sync_copy(counts.at[wid], cntv)
    cnt = jnp.max(cntv[pl.ds(0, 16)])
    nch = (cnt + SUB - 1) // SUB
    nq = jnp.maximum((nch + NBUF - 1) // NBUF, 1)

    def z_body(i, carry):
        for k2 in range(D // 16):
            rows_in[0, i, pl.ds(k2 * 16, 16)] = zero16
        return carry

    lax.fori_loop(0, SUB, z_body, 0)
    rstart = s * ROWS_PER_TILE
    for j in range(ROWS_PER_TILE // SUB):
        pltpu.sync_copy(rows_in.at[0], acc.at[pl.ds(rstart + j * SUB, SUB)])
    rem = ROWS_PER_TILE % SUB
    if rem:
        pltpu.sync_copy(rows_in.at[0, pl.ds(0, rem)],
                        acc.at[pl.ds(rstart + (ROWS_PER_TILE // SUB) * SUB, rem)])
    plsc.subcore_barrier()

    def issue_loads(i, q):
        pltpu.async_copy(col_c.at[wid, pl.ds(pl.multiple_of(i * SUB, SUB), SUB)], colv.at[q],
                         sem_i.at[q])
        pltpu.async_copy(row_c.at[wid, pl.ds(pl.multiple_of(i * SUB, SUB), SUB)], rowv.at[q],
                         sem_i.at[q])
        pltpu.async_copy(wt_c.at[wid, pl.ds(pl.multiple_of(i * SUB, SUB), SUB)], wtv.at[q],
                         sem_i.at[q])

    def wait_loads(q):
        pltpu.make_async_copy(col_c.at[0, pl.ds(0, SUB)], colv.at[q],
                              sem_i.at[q]).wait()
        pltpu.make_async_copy(row_c.at[0, pl.ds(0, SUB)], rowv.at[q],
                              sem_i.at[q]).wait()
        pltpu.make_async_copy(wt_c.at[0, pl.ds(0, SUB)], wtv.at[q],
                              sem_i.at[q]).wait()

    def issue_gather(q, b):
        pltpu.async_copy(table.at[colv.at[q]], rows_in.at[b], sem_g.at[b])

    def wait_gather(q, b):
        pltpu.make_async_copy(table.at[colv.at[q]], rows_in.at[b],
                              sem_g.at[b]).wait()

    def issue_scatter(q, b):
        pltpu.async_copy(rows_in.at[b], acc.at[rowv.at[q]], sem_s.at[b],
                         add=True)

    def wait_scatter(q, b):
        pltpu.make_async_copy(rows_in.at[b], acc.at[rowv.at[q]],
                              sem_s.at[b]).wait()

    def compute(q, b):
        qv = jnp.full((16,), q, jnp.int32)

        @plsc.parallel_loop(0, SUB, 1, unroll=8)
        def _(e):
            ev = jnp.full((16,), e, jnp.int32)
            wv = plsc.load_gather(wtv, [qv, ev])
            for k2 in range(D // 16):
                sl = pl.ds(k2 * 16, 16)
                rows_in[b, e, sl] = rows_in[b, e, sl] * wv

    def step(i, u, iq):
        b = u % NROW
        bp = (u + 2) % NROW
        q = u
        qn2 = (u + 2) % NBUF
        qn3 = (u + 3) % NBUF
        wait_gather(q, b)
        if u == 0:
            @pl.when(iq > 0)
            def _():
                wait_scatter(NBUF - 1, bp)
        else:
            wait_scatter(u - 1, bp)
        if u < 3:
            issue_loads(i + 3, qn3)
        else:
            @pl.when(iq < nq - 1)
            def _():
                issue_loads(i + 3, qn3)
        if u < 4:
            wait_loads(qn2)
            issue_gather(qn2, bp)
        else:
            @pl.when(iq < nq - 1)
            def _():
                wait_loads(qn2)
                issue_gather(qn2, bp)
        compute(q, b)
        issue_scatter(q, b)

    issue_loads(0, 0)
    issue_loads(1, 1)
    issue_loads(2, 2)
    wait_loads(0)
    issue_gather(0, 0)
    wait_loads(1)
    issue_gather(1, 1)

    def six_body(iq, carry):
        for u in range(NBUF):
            step(iq * NBUF + u, u, iq)
        return carry

    lax.fori_loop(0, nq, six_body, 0)
    wait_scatter(NBUF - 1, NROW - 1)
    plsc.subcore_barrier()

    @pl.when(s < NS - 1)
    def _():
        pltpu.sync_copy(acc.at[pl.ds(rstart, ROWS_PER_TILE)],
                        out.at[pl.ds(node_base + rstart, ROWS_PER_TILE)])

    @pl.when(s == NS - 1)
    def _():
        pltpu.sync_copy(acc.at[pl.ds(rstart, LAST_ROWS)],
                        out.at[pl.ds(node_base + rstart, LAST_ROWS)])


@jax.jit
def _propagate(table0, col1, row1, wt1):
    mesh = plsc.VectorSubcoreMesh(core_axis_name="c", subcore_axis_name="s",
                                  num_cores=NC, num_subcores=NS)
    cparams = pltpu.CompilerParams(use_tc_tiling_on_sc=False,
                                   needs_layout_passes=False)
    part = pl.kernel(
        _part_body,
        out_type=(
            jax.ShapeDtypeStruct((NC * NS, CAP), jnp.int32),
            jax.ShapeDtypeStruct((NC * NS, CAP), jnp.int32),
            jax.ShapeDtypeStruct((NC * NS, CAP), jnp.float32),
            jax.ShapeDtypeStruct((NC * NS, 16), jnp.int32),
        ),
        mesh=mesh,
        compiler_params=cparams,
        scratch_types=[
            pltpu.VMEM((B_LOAD,), jnp.int32),
            pltpu.VMEM((B_LOAD,), jnp.int32),
            pltpu.VMEM((B_LOAD,), jnp.float32),
            pltpu.VMEM((STG,), jnp.int32),
            pltpu.VMEM((STG,), jnp.int32),
            pltpu.VMEM((STG,), jnp.float32),
            pltpu.VMEM((16,), jnp.int32),
        ],
    )
    col_c, row_c, wt_c, counts = part(col1, row1, wt1)

    layer = pl.kernel(
        _layer_body,
        out_type=jax.ShapeDtypeStruct((N_NODES, D), jnp.float32),
        mesh=mesh,
        compiler_params=cparams,
        scratch_types=[
            pltpu.VMEM_SHARED((ACC_ROWS, D), jnp.float32),
            pltpu.VMEM((NBUF, SUB), jnp.int32),
            pltpu.VMEM((NBUF, SUB), jnp.int32),
            pltpu.VMEM((NBUF, SUB), jnp.float32),
            pltpu.VMEM((NROW, SUB, D), jnp.float32),
            pltpu.VMEM((16,), jnp.int32),
            pltpu.SemaphoreType.DMA((NBUF,)),
            pltpu.SemaphoreType.DMA((NROW,)),
            pltpu.SemaphoreType.DMA((NROW,)),
        ],
    )
    e1 = layer(table0, col_c, row_c, wt_c, counts)
    e2 = layer(e1, col_c, row_c, wt_c, counts)
    e3 = layer(e2, col_c, row_c, wt_c, counts)
    return e1, e2, e3


def _mean_body(e0, e1, e2, e3, o):
    o[...] = (e0[...] + e1[...] + e2[...] + e3[...]) * 0.25


def kernel(user_emb, item_emb, edge_index, edge_weight):
    row = edge_index[0].astype(jnp.int32)
    col = edge_index[1].astype(jnp.int32)
    n_edges = row.shape[0]
    step = NS * B_LOAD
    e_pad = ((n_edges + step - 1) // step) * step
    pad = e_pad - n_edges
    col_p = jnp.pad(col, (0, pad))
    row_p = jnp.pad(row, (0, pad), constant_values=-1)
    wt_p = jnp.pad(edge_weight, (0, pad))

    table0 = jnp.concatenate([user_emb, item_emb], axis=0)
    e1, e2, e3 = _propagate(table0, col_p, row_p, wt_p)

    blk = 1000
    spec = pl.BlockSpec((blk, D), lambda i: (i, 0))
    final = pl.pallas_call(
        _mean_body,
        grid=(N_NODES // blk,),
        in_specs=[spec] * 4,
        out_specs=spec,
        out_shape=jax.ShapeDtypeStruct((N_NODES, D), jnp.float32),
    )(table0, e1, e2, e3)
    return (final[:N_USERS], final[N_USERS:])


# speedup vs baseline: 1.0334x; 1.0334x over previous
import functools

import jax
import jax.numpy as jnp
from jax import lax
from jax.experimental import pallas as pl
from jax.experimental.pallas import tpu as pltpu
from jax.experimental.pallas import tpu_sc as plsc

N_USERS = 25000
N_ITEMS = 25000
N_NODES = N_USERS + N_ITEMS
N_LAYERS = 3
D = 64

NC = 2
NS = 16
HALF = N_NODES // NC
ROWS_PER_TILE = 1568
ACC_ROWS = ROWS_PER_TILE * NS
DUMMY = HALF
LAST_ROWS = HALF - (NS - 1) * ROWS_PER_TILE

SUB = 128
NBUF = 6
NROW = 3

B_LOAD = 7168
FLUSH = 1024
STG = FLUSH + 32
D_APP = 896
CAP = 52224


def _part_body(col1, row1, wt1, col_c, row_c, wt_c, counts,
               ldc, ldr, ldw, stc, st_r, stw, cntv):
    c = lax.axis_index("c")
    s = lax.axis_index("s")
    node_base = c * HALF
    span = col1.shape[0] // NS
    tbase = s * span
    wid = c * NS + s

    def flush_maybe(pos, off):
        def do_flush(args):
            pos, off = args
            pltpu.sync_copy(stc.at[pl.ds(0, FLUSH)],
                            col_c.at[wid, pl.ds(pl.multiple_of(off, FLUSH), FLUSH)])
            pltpu.sync_copy(st_r.at[pl.ds(0, FLUSH)],
                            row_c.at[wid, pl.ds(pl.multiple_of(off, FLUSH), FLUSH)])
            pltpu.sync_copy(stw.at[pl.ds(0, FLUSH)],
                            wt_c.at[wid, pl.ds(pl.multiple_of(off, FLUSH), FLUSH)])
            stc[pl.ds(0, 16)] = stc[pl.ds(FLUSH, 16)]
            st_r[pl.ds(0, 16)] = st_r[pl.ds(FLUSH, 16)]
            stw[pl.ds(0, 16)] = stw[pl.ds(FLUSH, 16)]
            return (pos - FLUSH, off + FLUSH)

        return lax.cond(pos >= FLUSH, do_flush, lambda a: a, (pos, off))

    def blk(ib, carry):
        pltpu.sync_copy(col1.at[pl.ds(tbase + ib * B_LOAD, B_LOAD)], ldc)
        pltpu.sync_copy(row1.at[pl.ds(tbase + ib * B_LOAD, B_LOAD)], ldr)
        pltpu.sync_copy(wt1.at[pl.ds(tbase + ib * B_LOAD, B_LOAD)], ldw)

        def grp(g, pc):
            pos, off = pc
            sl = pl.ds(g * 16, 16)
            loc = ldr[sl] - node_base
            m = (loc >= 0) & (loc < HALF)
            plsc.store_compressed(stc.at[pl.ds(pos, 16)], ldc[sl], mask=m)
            plsc.store_compressed(st_r.at[pl.ds(pos, 16)], loc, mask=m)
            plsc.store_compressed(stw.at[pl.ds(pos, 16)], ldw[sl], mask=m)
            pos = pos + jnp.sum(m.astype(jnp.int32))
            return flush_maybe(pos, off)

        return lax.fori_loop(0, B_LOAD // 16, grp, carry)

    pos, off = lax.fori_loop(0, span // B_LOAD, blk,
                             (jnp.int32(0), jnp.int32(0)))

    cnt = off + pos
    cntv[pl.ds(0, 16)] = jnp.full((16,), cnt, jnp.int32)
    pltpu.sync_copy(cntv, counts.at[wid])

    dumc = jnp.zeros((16,), jnp.int32)
    dumr = jnp.full((16,), DUMMY, jnp.int32)
    dumw = jnp.zeros((16,), jnp.float32)

    def app(g, pc):
        pos, off = pc
        stc[pl.ds(pos, 16)] = dumc
        st_r[pl.ds(pos, 16)] = dumr
        stw[pl.ds(pos, 16)] = dumw
        return flush_maybe(pos + 16, off)

    pos, off = lax.fori_loop(0, D_APP // 16, app, (pos, off))
    pltpu.sync_copy(stc.at[pl.ds(0, FLUSH)], col_c.at[wid, pl.ds(pl.multiple_of(off, FLUSH), FLUSH)])
    pltpu.sync_copy(st_r.at[pl.ds(0, FLUSH)], row_c.at[wid, pl.ds(pl.multiple_of(off, FLUSH), FLUSH)])
    pltpu.sync_copy(stw.at[pl.ds(0, FLUSH)], wt_c.at[wid, pl.ds(pl.multiple_of(off, FLUSH), FLUSH)])


def _layer_body(table, col_c, row_c, wt_c, counts, out,
                acc, colv, rowv, wtv, rows_in, cntv,
                sem_i, sem_g, sem_s):
    c = lax.axis_index("c")
    s = lax.axis_index("s")
    node_base = c * HALF
    wid = c * NS + s
    zero16 = jnp.zeros((16,), jnp.float32)

    pltpu.sync_copy(counts.at[wid], cntv)
    cnt = jnp.max(cntv[pl.ds(0, 16)])
    nch = (cnt + SUB - 1) // SUB
    nq = jnp.maximum((nch + NBUF - 1) // NBUF, 1)

    def z_body(i, carry):
        for k2 in range(D // 16):
            rows_in[0, i, pl.ds(k2 * 16, 16)] = zero16
        return carry

    lax.fori_loop(0, SUB, z_body, 0)
    rstart = s * ROWS_PER_TILE
    for j in range(ROWS_PER_TILE // SUB):
        pltpu.sync_copy(rows_in.at[0], acc.at[pl.ds(rstart + j * SUB, SUB)])
    rem = ROWS_PER_TILE % SUB
    if rem:
        pltpu.sync_copy(rows_in.at[0, pl.ds(0, rem)],
                        acc.at[pl.ds(rstart + (ROWS_PER_TILE // SUB) * SUB, rem)])
    plsc.subcore_barrier()

    def issue_loads(i, q):
        pltpu.async_copy(col_c.at[wid, pl.ds(pl.multiple_of(i * SUB, SUB), SUB)], colv.at[q],
                         sem_i.at[q])
        pltpu.async_copy(row_c.at[wid, pl.ds(pl.multiple_of(i * SUB, SUB), SUB)], rowv.at[q],
                         sem_i.at[q])
        pltpu.async_copy(wt_c.at[wid, pl.ds(pl.multiple_of(i * SUB, SUB), SUB)], wtv.at[q],
                         sem_i.at[q])

    def wait_loads(q):
        pltpu.make_async_copy(col_c.at[0, pl.ds(0, SUB)], colv.at[q],
                              sem_i.at[q]).wait()
        pltpu.make_async_copy(row_c.at[0, pl.ds(0, SUB)], rowv.at[q],
                              sem_i.at[q]).wait()
        pltpu.make_async_copy(wt_c.at[0, pl.ds(0, SUB)], wtv.at[q],
                              sem_i.at[q]).wait()

    def issue_gather(q, b):
        pltpu.async_copy(table.at[colv.at[q]], rows_in.at[b], sem_g.at[b])

    def wait_gather(q, b):
        pltpu.make_async_copy(table.at[colv.at[q]], rows_in.at[b],
                              sem_g.at[b]).wait()

    def issue_scatter(q, b):
        pltpu.async_copy(rows_in.at[b], acc.at[rowv.at[q]], sem_s.at[b],
                         add=True)

    def wait_scatter(q, b):
        pltpu.make_async_copy(rows_in.at[b], acc.at[rowv.at[q]],
                              sem_s.at[b]).wait()

    def compute(q, b):
        qv = jnp.full((16,), q, jnp.int32)

        @plsc.parallel_loop(0, SUB, 1, unroll=4)
        def _(e):
            ev = jnp.full((16,), e, jnp.int32)
            wv = plsc.load_gather(wtv, [qv, ev])
            for k2 in range(D // 16):
                sl = pl.ds(k2 * 16, 16)
                rows_in[b, e, sl] = rows_in[b, e, sl] * wv

    def step(i, u, iq):
        b = u % NROW
        bp = (u + 2) % NROW
        q = u
        qn2 = (u + 2) % NBUF
        qn3 = (u + 3) % NBUF
        wait_gather(q, b)
        if u == 0:
            @pl.when(iq > 0)
            def _():
                wait_scatter(NBUF - 1, bp)
        else:
            wait_scatter(u - 1, bp)
        if u < 3:
            issue_loads(i + 3, qn3)
        else:
            @pl.when(iq < nq - 1)
            def _():
                issue_loads(i + 3, qn3)
        if u < 4:
            wait_loads(qn2)
            issue_gather(qn2, bp)
        else:
            @pl.when(iq < nq - 1)
            def _():
                wait_loads(qn2)
                issue_gather(qn2, bp)
        compute(q, b)
        issue_scatter(q, b)

    issue_loads(0, 0)
    issue_loads(1, 1)
    issue_loads(2, 2)
    wait_loads(0)
    issue_gather(0, 0)
    wait_loads(1)
    issue_gather(1, 1)

    def six_body(iq, carry):
        for u in range(NBUF):
            step(iq * NBUF + u, u, iq)
        return carry

    lax.fori_loop(0, nq, six_body, 0)
    wait_scatter(NBUF - 1, NROW - 1)
    plsc.subcore_barrier()

    @pl.when(s < NS - 1)
    def _():
        pltpu.sync_copy(acc.at[pl.ds(rstart, ROWS_PER_TILE)],
                        out.at[pl.ds(node_base + rstart, ROWS_PER_TILE)])

    @pl.when(s == NS - 1)
    def _():
        pltpu.sync_copy(acc.at[pl.ds(rstart, LAST_ROWS)],
                        out.at[pl.ds(node_base + rstart, LAST_ROWS)])


@jax.jit
def _propagate(table0, col1, row1, wt1):
    mesh = plsc.VectorSubcoreMesh(core_axis_name="c", subcore_axis_name="s",
                                  num_cores=NC, num_subcores=NS)
    cparams = pltpu.CompilerParams(use_tc_tiling_on_sc=False,
                                   needs_layout_passes=False)
    part = pl.kernel(
        _part_body,
        out_type=(
            jax.ShapeDtypeStruct((NC * NS, CAP), jnp.int32),
            jax.ShapeDtypeStruct((NC * NS, CAP), jnp.int32),
            jax.ShapeDtypeStruct((NC * NS, CAP), jnp.float32),
            jax.ShapeDtypeStruct((NC * NS, 16), jnp.int32),
        ),
        mesh=mesh,
        compiler_params=cparams,
        scratch_types=[
            pltpu.VMEM((B_LOAD,), jnp.int32),
            pltpu.VMEM((B_LOAD,), jnp.int32),
            pltpu.VMEM((B_LOAD,), jnp.float32),
            pltpu.VMEM((STG,), jnp.int32),
            pltpu.VMEM((STG,), jnp.int32),
            pltpu.VMEM((STG,), jnp.float32),
            pltpu.VMEM((16,), jnp.int32),
        ],
    )
    col_c, row_c, wt_c, counts = part(col1, row1, wt1)

    layer = pl.kernel(
        _layer_body,
        out_type=jax.ShapeDtypeStruct((N_NODES, D), jnp.float32),
        mesh=mesh,
        compiler_params=cparams,
        scratch_types=[
            pltpu.VMEM_SHARED((ACC_ROWS, D), jnp.float32),
            pltpu.VMEM((NBUF, SUB), jnp.int32),
            pltpu.VMEM((NBUF, SUB), jnp.int32),
            pltpu.VMEM((NBUF, SUB), jnp.float32),
            pltpu.VMEM((NROW, SUB, D), jnp.float32),
            pltpu.VMEM((16,), jnp.int32),
            pltpu.SemaphoreType.DMA((NBUF,)),
            pltpu.SemaphoreType.DMA((NROW,)),
            pltpu.SemaphoreType.DMA((NROW,)),
        ],
    )
    e1 = layer(table0, col_c, row_c, wt_c, counts)
    e2 = layer(e1, col_c, row_c, wt_c, counts)
    e3 = layer(e2, col_c, row_c, wt_c, counts)
    return e1, e2, e3


def _mean_body(e0, e1, e2, e3, o):
    o[...] = (e0[...] + e1[...] + e2[...] + e3[...]) * 0.25


def kernel(user_emb, item_emb, edge_index, edge_weight):
    row = edge_index[0].astype(jnp.int32)
    col = edge_index[1].astype(jnp.int32)
    n_edges = row.shape[0]
    step = NS * B_LOAD
    e_pad = ((n_edges + step - 1) // step) * step
    pad = e_pad - n_edges
    col_p = jnp.pad(col, (0, pad))
    row_p = jnp.pad(row, (0, pad), constant_values=-1)
    wt_p = jnp.pad(edge_weight, (0, pad))

    table0 = jnp.concatenate([user_emb, item_emb], axis=0)
    e1, e2, e3 = _propagate(table0, col_p, row_p, wt_p)

    blk = 1000
    spec = pl.BlockSpec((blk, D), lambda i: (i, 0))
    final = pl.pallas_call(
        _mean_body,
        grid=(N_NODES // blk,),
        in_specs=[spec] * 4,
        out_specs=spec,
        out_shape=jax.ShapeDtypeStruct((N_NODES, D), jnp.float32),
    )(table0, e1, e2, e3)
    return (final[:N_USERS], final[N_USERS:])
